# baseline (speedup 1.0000x reference)
import functools

import jax
import jax.numpy as jnp
from jax import lax
from jax.experimental import pallas as pl
from jax.experimental.pallas import tpu as pltpu
from jax.experimental.pallas import tpu_sc as plsc


def kernel(x_user, x_track_pos, x_track_neg, user_mat, track_mat):
    B = x_user.shape[0]
    V, D = user_mat.shape
    info = plsc.get_sparse_core_info()
    NW = info.num_cores * info.num_subcores
    L = info.num_lanes
    b = B // NW
    K = 8

    ut = user_mat.T
    tt = track_mat.T

    mesh = plsc.VectorSubcoreMesh(core_axis_name="c", subcore_axis_name="s")
    out_sds = jax.ShapeDtypeStruct((D, B), jnp.float32)

    @functools.partial(
        pl.kernel,
        mesh=mesh,
        out_type=(out_sds, out_sds, out_sds),
        scratch_types=(
            [pltpu.VMEM((b,), jnp.int32),
             pltpu.VMEM((K * D, 128), jnp.float32),
             pltpu.VMEM((D, b), jnp.float32)]
            + [pltpu.SemaphoreType.DMA for _ in range(2 * K)]
        ),
        compiler_params=pltpu.CompilerParams(needs_layout_passes=False),
    )
    def gather3(xu, xp, xn, ut_r, tt_r, out_u, out_p, out_n,
                idx_v, ring, vals, *sems):
        wid = lax.axis_index("s") * info.num_cores + lax.axis_index("c")
        base = wid * b
        lanes = lax.iota(jnp.int32, L)

        def one(x_hbm, tf, out_hbm):
            pltpu.sync_copy(x_hbm.at[pl.ds(base, b)], idx_v)

            H = D // 2

            def fire(xi, slot):
                blk = pl.multiple_of((xi >> 7) * 128, 128)
                pltpu.async_copy(
                    tf.at[pl.ds(0, H), pl.ds(blk, 128)],
                    ring.at[pl.ds(slot * D, H), pl.ds(0, 128)],
                    sems[2 * slot])
                pltpu.async_copy(
                    tf.at[pl.ds(H, H), pl.ds(blk, 128)],
                    ring.at[pl.ds(slot * D + H, H), pl.ds(0, 128)],
                    sems[2 * slot + 1])

            def drain_extract(xi, i_dst, slot):
                pltpu.make_async_copy(
                    tf.at[pl.ds(0, H), pl.ds(0, 128)],
                    ring.at[pl.ds(slot * D, H), pl.ds(0, 128)],
                    sems[2 * slot]).wait()
                pltpu.make_async_copy(
                    tf.at[pl.ds(0, H), pl.ds(0, 128)],
                    ring.at[pl.ds(slot * D + H, H), pl.ds(0, 128)],
                    sems[2 * slot + 1]).wait()
                col = jnp.broadcast_to(xi & 127, (L,))
                dst = jnp.broadcast_to(i_dst, (L,))
                for t in range(D // L):
                    row = slot * D + t * L + lanes
                    v = plsc.load_gather(ring, [row, col])
                    plsc.store_scatter(vals, [t * L + lanes, dst], v)

            first = idx_v[pl.ds(0, L)]
            for s in range(K):
                fire(first[s], s)

            def body(g, _):
                ch = idx_v[pl.ds(g * K, 2 * K)]
                for s in range(K):
                    drain_extract(ch[s], g * K + s, s)
                    fire(ch[K + s], s)
                return 0

            lax.fori_loop(0, b // K - 1, body, 0)
            last = idx_v[pl.ds(b - 2 * K, 2 * K)]
            for s in range(K):
                drain_extract(last[K + s], b - K + s, s)
            pltpu.sync_copy(vals, out_hbm.at[pl.ds(0, D), pl.ds(base, b)])

        one(xu, ut_r, out_u)
        one(xp, tt_r, out_p)
        one(xn, tt_r, out_n)

    u_t, p_t, n_t = gather3(x_user, x_track_pos, x_track_neg, ut, tt)
    return (u_t.T, p_t.T, n_t.T)


# speedup vs baseline: 1.1195x; 1.1195x over previous
import functools

import jax
import jax.numpy as jnp
from jax import lax
from jax.experimental import pallas as pl
from jax.experimental.pallas import tpu as pltpu
from jax.experimental.pallas import tpu_sc as plsc


def kernel(x_user, x_track_pos, x_track_neg, user_mat, track_mat):
    B = x_user.shape[0]
    V, D = user_mat.shape
    info = plsc.get_sparse_core_info()
    NW = info.num_cores * info.num_subcores
    L = info.num_lanes
    b = B // NW
    K = 8

    ut = user_mat.T
    tt = track_mat.T

    mesh = plsc.VectorSubcoreMesh(core_axis_name="c", subcore_axis_name="s")
    out_sds = jax.ShapeDtypeStruct((D, B), jnp.float32)

    @functools.partial(
        pl.kernel,
        mesh=mesh,
        out_type=(out_sds, out_sds, out_sds),
        scratch_types=(
            [pltpu.VMEM((b,), jnp.int32),
             pltpu.VMEM((K * D, 128), jnp.float32),
             pltpu.VMEM((D, b), jnp.float32)]
            + [pltpu.SemaphoreType.DMA for _ in range(K)]
        ),
        compiler_params=pltpu.CompilerParams(needs_layout_passes=False),
    )
    def gather3(xu, xp, xn, ut_r, tt_r, out_u, out_p, out_n,
                idx_v, ring, vals, *sems):
        wid = lax.axis_index("s") * info.num_cores + lax.axis_index("c")
        base = wid * b
        lanes = lax.iota(jnp.int32, L)

        def one(x_hbm, tf, out_hbm):
            pltpu.sync_copy(x_hbm.at[pl.ds(base, b)], idx_v)

            def fire(xi, slot):
                blk = pl.multiple_of((xi >> 7) * 128, 128)
                pltpu.async_copy(
                    tf.at[pl.ds(0, D), pl.ds(blk, 128)],
                    ring.at[pl.ds(slot * D, D), pl.ds(0, 128)],
                    sems[slot])

            def drain_extract(xi, i_dst, slot):
                pltpu.make_async_copy(
                    tf.at[pl.ds(0, D), pl.ds(0, 128)],
                    ring.at[pl.ds(slot * D, D), pl.ds(0, 128)],
                    sems[slot]).wait()
                col = jnp.broadcast_to(xi & 127, (L,))
                dst = jnp.broadcast_to(i_dst, (L,))
                for t in range(D // L):
                    row = slot * D + t * L + lanes
                    v = plsc.load_gather(ring, [row, col])
                    plsc.store_scatter(vals, [t * L + lanes, dst], v)

            first = idx_v[pl.ds(0, L)]
            for s in range(K):
                fire(first[s], s)

            def body(g, _):
                ch = idx_v[pl.ds(g * K, 2 * K)]
                for s in range(K):
                    drain_extract(ch[s], g * K + s, s)
                    fire(ch[K + s], s)
                return 0

            lax.fori_loop(0, b // K - 1, body, 0)
            last = idx_v[pl.ds(b - 2 * K, 2 * K)]
            for s in range(K):
                drain_extract(last[K + s], b - K + s, s)
            pltpu.sync_copy(vals, out_hbm.at[pl.ds(0, D), pl.ds(base, b)])

        one(xu, ut_r, out_u)
        one(xp, tt_r, out_p)
        one(xn, tt_r, out_n)

    u_t, p_t, n_t = gather3(x_user, x_track_pos, x_track_neg, ut, tt)
    return (u_t.T, p_t.T, n_t.T)


# speedup vs baseline: 1.1245x; 1.0044x over previous
import functools

import jax
import jax.numpy as jnp
from jax import lax
from jax.experimental import pallas as pl
from jax.experimental.pallas import tpu as pltpu
from jax.experimental.pallas import tpu_sc as plsc


def kernel(x_user, x_track_pos, x_track_neg, user_mat, track_mat):
    B = x_user.shape[0]
    V, D = user_mat.shape
    info = plsc.get_sparse_core_info()
    NW = info.num_cores * info.num_subcores
    L = info.num_lanes
    b = B // NW
    K = 8

    ut = user_mat.T
    tt = track_mat.T

    mesh = plsc.VectorSubcoreMesh(core_axis_name="c", subcore_axis_name="s")
    out_sds = jax.ShapeDtypeStruct((D, B), jnp.float32)

    @functools.partial(
        pl.kernel,
        mesh=mesh,
        out_type=(out_sds, out_sds, out_sds),
        scratch_types=(
            [pltpu.VMEM((b,), jnp.int32),
             pltpu.VMEM((K * D, 128), jnp.float32),
             pltpu.VMEM((D, b), jnp.float32)]
            + [pltpu.SemaphoreType.DMA for _ in range(K + 1)]
        ),
        compiler_params=pltpu.CompilerParams(needs_layout_passes=False),
    )
    def gather3(xu, xp, xn, ut_r, tt_r, out_u, out_p, out_n,
                idx_v, ring, vals, *sems):
        wid = lax.axis_index("s") * info.num_cores + lax.axis_index("c")
        base = wid * b
        lanes = lax.iota(jnp.int32, L)

        sem_o = sems[K]

        def one(x_hbm, tf, out_hbm, wb_prev):
            pltpu.sync_copy(x_hbm.at[pl.ds(base, b)], idx_v)

            def fire(xi, slot):
                blk = pl.multiple_of((xi >> 7) * 128, 128)
                pltpu.async_copy(
                    tf.at[pl.ds(0, D), pl.ds(blk, 128)],
                    ring.at[pl.ds(slot * D, D), pl.ds(0, 128)],
                    sems[slot])

            def drain_extract(xi, i_dst, slot):
                pltpu.make_async_copy(
                    tf.at[pl.ds(0, D), pl.ds(0, 128)],
                    ring.at[pl.ds(slot * D, D), pl.ds(0, 128)],
                    sems[slot]).wait()
                col = jnp.broadcast_to(xi & 127, (L,))
                dst = jnp.broadcast_to(i_dst, (L,))
                for t in range(D // L):
                    row = slot * D + t * L + lanes
                    v = plsc.load_gather(ring, [row, col])
                    plsc.store_scatter(vals, [t * L + lanes, dst], v)

            first = idx_v[pl.ds(0, L)]
            for s in range(K):
                fire(first[s], s)
            if wb_prev is not None:
                wb_prev.wait()

            def body(g, _):
                ch = idx_v[pl.ds(g * K, 2 * K)]
                for s in range(K):
                    drain_extract(ch[s], g * K + s, s)
                    fire(ch[K + s], s)
                return 0

            lax.fori_loop(0, b // K - 1, body, 0)
            last = idx_v[pl.ds(b - 2 * K, 2 * K)]
            for s in range(K):
                drain_extract(last[K + s], b - K + s, s)
            return pltpu.async_copy(
                vals, out_hbm.at[pl.ds(0, D), pl.ds(base, b)], sem_o)

        wb = one(xu, ut_r, out_u, None)
        wb = one(xp, tt_r, out_p, wb)
        wb = one(xn, tt_r, out_n, wb)
        wb.wait()

    u_t, p_t, n_t = gather3(x_user, x_track_pos, x_track_neg, ut, tt)
    return (u_t.T, p_t.T, n_t.T)
